# baseline (speedup 1.0000x reference)
import jax
import jax.numpy as jnp
from jax.experimental import pallas as pl
from jax.experimental.pallas import tpu as pltpu

ISIZE = 224
PSIZE = 30
ROW0 = 172
COL0 = 47
RB = 8
RBI0 = 21
NRB = 5
BB = 16


def _band_kernel(x_ref, patch_ref, out_ref, canvas_ref):
    j = pl.program_id(1)

    @pl.when(pl.program_id(0) + j == 0)
    def _build_canvas():
        canvas_ref[...] = jnp.zeros_like(canvas_ref)
        canvas_ref[:, :, ROW0 - RBI0 * RB:ROW0 - RBI0 * RB + PSIZE,
                   COL0:COL0 + PSIZE] = patch_ref[...]

    blk = x_ref[...]
    out_ref[...] = blk + canvas_ref[:, :, pl.ds(j * RB, RB), :]


def kernel(x, patch):
    batch = x.shape[0]
    grid = (batch // BB, NRB)
    return pl.pallas_call(
        _band_kernel,
        grid=grid,
        in_specs=[
            pl.BlockSpec((BB, 3, RB, 128), lambda i, j: (i, 0, RBI0 + j, 0)),
            pl.BlockSpec((1, 3, PSIZE, PSIZE), lambda i, j: (0, 0, 0, 0)),
        ],
        out_specs=pl.BlockSpec((BB, 3, RB, 128), lambda i, j: (i, 0, RBI0 + j, 0)),
        out_shape=jax.ShapeDtypeStruct(x.shape, x.dtype),
        input_output_aliases={0: 0},
        scratch_shapes=[pltpu.VMEM((1, 3, NRB * RB, 128), jnp.float32)],
    )(x, patch)
